# baseline (speedup 1.0000x reference)
import functools
import jax
import jax.numpy as jnp
import numpy as np
from jax import lax
from jax.experimental import pallas as pl
from jax.experimental.pallas import tpu as pltpu
from jax.experimental.pallas import tpu_sc as plsc

HEADS = 4

_NC, _NS, _LANES = 2, 16, 16
_NTILES = _NC * _NS
_CHUNK = 256
_CHUNKS_PER_TILE = 21
_EDGES_PER_TILE = _CHUNK * _CHUNKS_PER_TILE
_E_PAD = _EDGES_PER_TILE * _NTILES
_NPAD = 10240
_ROWS_PER_TILE = _NPAD // _NS


def _layer_mm_body(x_ref, w_ref, asrc_ref, adst_ref, h_ref, es_ref, ed_ref,
                   em_ref):
    h = jnp.dot(x_ref[...], w_ref[...], preferred_element_type=jnp.float32)
    h_ref[...] = h
    es = jnp.dot(h, asrc_ref[...], preferred_element_type=jnp.float32)
    ed = jnp.dot(h, adst_ref[...], preferred_element_type=jnp.float32)
    es_ref[...] = es
    ed_ref[...] = ed

    @pl.when(pl.program_id(0) == 0)
    def _():
        em_ref[...] = jnp.full_like(em_ref[...], -jnp.inf)

    em_ref[...] = jnp.maximum(em_ref[...], jnp.max(es, axis=0, keepdims=True))


def _layer_mm(x, W, asrc_bd, adst_bd, n_rows, row_blk):
    din = x.shape[1]
    dout = W.shape[1]
    grid = n_rows // row_blk
    return pl.pallas_call(
        _layer_mm_body,
        grid=(grid,),
        in_specs=[
            pl.BlockSpec((row_blk, din), lambda i: (i, 0)),
            pl.BlockSpec((din, dout), lambda i: (0, 0)),
            pl.BlockSpec((dout, HEADS), lambda i: (0, 0)),
            pl.BlockSpec((dout, HEADS), lambda i: (0, 0)),
        ],
        out_specs=[
            pl.BlockSpec((row_blk, dout), lambda i: (i, 0)),
            pl.BlockSpec((row_blk, HEADS), lambda i: (i, 0)),
            pl.BlockSpec((row_blk, HEADS), lambda i: (i, 0)),
            pl.BlockSpec((1, HEADS), lambda i: (0, 0)),
        ],
        out_shape=[
            jax.ShapeDtypeStruct((n_rows, dout), jnp.float32),
            jax.ShapeDtypeStruct((n_rows, HEADS), jnp.float32),
            jax.ShapeDtypeStruct((n_rows, HEADS), jnp.float32),
            jax.ShapeDtypeStruct((1, HEADS), jnp.float32),
        ],
    )(x, W, asrc_bd, adst_bd)


def _leaky(x):
    return jnp.where(x > 0, x, 0.2 * x)


def _c_body(es_ref, ed_ref, em_ref, p_ref):
    es = es_ref[...]
    ed = ed_ref[...]
    t = em_ref[0, :][None, :] + ed
    u = es + ed
    c = jnp.maximum(_leaky(t), _leaky(u))
    z = jnp.zeros_like(es)
    p_ref[...] = jnp.concatenate([es, ed, c, z], axis=1)


def _c_kernel(es, ed, em):
    n = es.shape[0]
    return pl.pallas_call(
        _c_body,
        out_shape=jax.ShapeDtypeStruct((n, 4 * HEADS), jnp.float32),
    )(es, ed, em)


def _post_body(nheads, p0_ref, p1_ref, dinv_ref, b_ref, out_ref):
    ch = p0_ref.shape[1] // nheads
    scale = jnp.repeat(dinv_ref[...], ch, axis=1)
    v = (p0_ref[...] + p1_ref[...]) * scale + b_ref[...]
    out_ref[...] = jnp.where(v > 0, v, jnp.exp(jnp.minimum(v, 0.0)) - 1.0)


def _post_kernel(p0, p1, dinv, b, n_rows, row_blk):
    d = p0.shape[1]
    nheads = dinv.shape[1]
    grid = n_rows // row_blk
    return pl.pallas_call(
        functools.partial(_post_body, nheads),
        grid=(grid,),
        in_specs=[
            pl.BlockSpec((row_blk, d), lambda i: (i, 0)),
            pl.BlockSpec((row_blk, d), lambda i: (i, 0)),
            pl.BlockSpec((row_blk, nheads), lambda i: (i, 0)),
            pl.BlockSpec((1, d), lambda i: (0, 0)),
        ],
        out_specs=pl.BlockSpec((row_blk, d), lambda i: (i, 0)),
        out_shape=jax.ShapeDtypeStruct((n_rows, d), jnp.float32),
    )(p0, p1, dinv, b.reshape(1, d))


def _mean_row_body(h_ref, wf1a_ref, bf1_ref, row_ref):
    hm = jnp.mean(h_ref[...], axis=0, keepdims=True)
    row_ref[...] = (
        jnp.dot(hm, wf1a_ref[...], preferred_element_type=jnp.float32)
        + bf1_ref[...]
    )


def _mean_row_kernel(h, wf1a, bf1):
    f = wf1a.shape[1]
    return pl.pallas_call(
        _mean_row_body,
        out_shape=jax.ShapeDtypeStruct((1, f), jnp.float32),
    )(h, wf1a, bf1.reshape(1, f))


def _mlp_body(ea_ref, row_ref, wf1b_ref, wf2_ref, bf2_ref, wf3_ref, bf3_ref,
              out_ref):
    z = jnp.dot(ea_ref[...], wf1b_ref[...], preferred_element_type=jnp.float32)
    z = jnp.maximum(z + row_ref[...], 0.0)
    z = jnp.dot(z, wf2_ref[...], preferred_element_type=jnp.float32)
    z = jnp.maximum(z + bf2_ref[...], 0.0)
    z = jnp.dot(z, wf3_ref[...], preferred_element_type=jnp.float32)
    out_ref[...] = z + bf3_ref[...]


def _mlp_kernel(ea, row, wf1b, wf2, bf2, wf3, bf3, row_blk):
    e, de = ea.shape
    f1 = row.shape[1]
    f2 = wf2.shape[1]
    grid = e // row_blk
    return pl.pallas_call(
        _mlp_body,
        grid=(grid,),
        in_specs=[
            pl.BlockSpec((row_blk, de), lambda i: (i, 0)),
            pl.BlockSpec((1, f1), lambda i: (0, 0)),
            pl.BlockSpec((de, f1), lambda i: (0, 0)),
            pl.BlockSpec((f1, f2), lambda i: (0, 0)),
            pl.BlockSpec((1, f2), lambda i: (0, 0)),
            pl.BlockSpec((f2, 1), lambda i: (0, 0)),
            pl.BlockSpec((1, 1), lambda i: (0, 0)),
        ],
        out_specs=pl.BlockSpec((row_blk, 1), lambda i: (i, 0)),
        out_shape=jax.ShapeDtypeStruct((e, 1), jnp.float32),
    )(ea, row, wf1b, wf2, bf2.reshape(1, f2), wf3, bf3.reshape(1, 1))


def _sc_mesh():
    return plsc.VectorSubcoreMesh(core_axis_name="c", subcore_axis_name="s")


_W = 4 * HEADS


def _vbcast(vec, k):
    idx = (lax.iota(jnp.int32, _LANES) >> 4) + k
    return vec.at[idx].get(mode="promise_in_bounds", unique_indices=False)


def _pass1_body(n_real_edges, src_h, dst_h, p_h, zeros_h,
                ee_h, den_h,
                idx_s, idx_d, psr, pdr, eer, den_sh, sem):
    cid = lax.axis_index("c")
    sid = lax.axis_index("s")
    wid = cid * _NS + sid
    tile_base = wid * _EDGES_PER_TILE

    @pl.when(sid == 0)
    def _():
        pltpu.sync_copy(zeros_h, den_sh)

    plsc.subcore_barrier()

    lanes = lax.iota(jnp.int32, _LANES)
    head = lanes & 3
    ed_idx = head + 4
    c_idx = head + 8

    def chunk(j, carry):
        base = tile_base + j * _CHUNK
        pltpu.sync_copy(src_h.at[pl.ds(base, _CHUNK)], idx_s)
        pltpu.sync_copy(dst_h.at[pl.ds(base, _CHUNK)], idx_d)
        pltpu.async_copy(p_h.at[idx_s], psr, sem).wait()
        pltpu.async_copy(p_h.at[idx_d], pdr, sem).wait()

        def edge4(gg, c2):
            for uu in range(4):
                e = gg * 4 + uu
                a = psr[e, :]
                b = pdr[e, :]
                ed_al = b.at[ed_idx].get(mode="promise_in_bounds")
                c_al = b.at[c_idx].get(mode="promise_in_bounds")
                u = a + ed_al
                lr = jnp.where(u > 0, u, 0.2 * u)
                t = jnp.exp(lr - c_al)
                m4 = jnp.where(lanes < 4, 1.0, 0.0)
                g = jnp.where(base + e < n_real_edges, 1.0, 0.0)
                eer[e, :] = t * (m4 * g)
            return c2

        lax.fori_loop(0, _CHUNK // 4, edge4, 0)
        pltpu.sync_copy(eer, ee_h.at[pl.ds(base, _CHUNK)])
        pltpu.sync_copy(eer, den_sh.at[idx_d], add=True)
        return carry

    lax.fori_loop(0, _CHUNKS_PER_TILE, chunk, 0)
    plsc.subcore_barrier()
    row0 = sid * _ROWS_PER_TILE
    pltpu.sync_copy(den_sh.at[pl.ds(row0, _ROWS_PER_TILE)],
                    den_h.at[pl.ds(cid * _NPAD + row0, _ROWS_PER_TILE)])


def _sc_pass1(src_p, dst_p, p, n_real_edges):
    kfn = pl.kernel(
        functools.partial(_pass1_body, n_real_edges),
        mesh=_sc_mesh(),
        out_type=[
            jax.ShapeDtypeStruct((_E_PAD, _W), jnp.float32),
            jax.ShapeDtypeStruct((_NC * _NPAD, _W), jnp.float32),
        ],
        scratch_types=[
            pltpu.VMEM((_CHUNK,), jnp.int32),
            pltpu.VMEM((_CHUNK,), jnp.int32),
            pltpu.VMEM((_CHUNK, _W), jnp.float32),
            pltpu.VMEM((_CHUNK, _W), jnp.float32),
            pltpu.VMEM((_CHUNK, _W), jnp.float32),
            pltpu.VMEM_SHARED((_NPAD, _W), jnp.float32),
            pltpu.SemaphoreType.DMA,
        ],
        compiler_params=pltpu.CompilerParams(use_tc_tiling_on_sc=False),
    )
    zeros = jnp.zeros((_NPAD, _W), jnp.float32)
    return kfn(src_p, dst_p, p, zeros)


def _pass2_body(ch, head_base, src_h, dst_h, ee_h, h_h, zeros_h,
                out_h,
                idx_s, idx_d, eer, hr, acc_sh, sem):
    cid = lax.axis_index("c")
    sid = lax.axis_index("s")
    wid = cid * _NS + sid
    tile_base = wid * _EDGES_PER_TILE
    d = hr.shape[1]
    vregs_per_row = d // _LANES

    @pl.when(sid == 0)
    def _():
        pltpu.sync_copy(zeros_h, acc_sh)

    plsc.subcore_barrier()

    def chunk(j, carry):
        base = tile_base + j * _CHUNK
        pltpu.sync_copy(src_h.at[pl.ds(base, _CHUNK)], idx_s)
        pltpu.sync_copy(dst_h.at[pl.ds(base, _CHUNK)], idx_d)
        pltpu.sync_copy(ee_h.at[pl.ds(base, _CHUNK)], eer)
        pltpu.async_copy(h_h.at[idx_s], hr, sem).wait()

        def edge4(g, c2):
            for u in range(4):
                e = g * 4 + u
                arow = eer[e, :]
                bcs = {}
                for jv in range(vregs_per_row):
                    hd = head_base + (jv * _LANES) // ch
                    if hd not in bcs:
                        bcs[hd] = _vbcast(arow, hd)
                    sl = pl.ds(jv * _LANES, _LANES)
                    hr[e, sl] = hr[e, sl] * bcs[hd]
            return c2

        lax.fori_loop(0, _CHUNK // 4, edge4, 0)
        pltpu.sync_copy(hr, acc_sh.at[idx_d], add=True)
        return carry

    lax.fori_loop(0, _CHUNKS_PER_TILE, chunk, 0)
    plsc.subcore_barrier()
    row0 = sid * _ROWS_PER_TILE
    pltpu.sync_copy(acc_sh.at[pl.ds(row0, _ROWS_PER_TILE)],
                    out_h.at[pl.ds(cid * _NPAD + row0, _ROWS_PER_TILE)])


def _sc_pass2(src_p, dst_p, ee, h, ch, head_base):
    d = h.shape[1]
    kfn = pl.kernel(
        functools.partial(_pass2_body, ch, head_base),
        mesh=_sc_mesh(),
        out_type=jax.ShapeDtypeStruct((_NC * _NPAD, d), jnp.float32),
        scratch_types=[
            pltpu.VMEM((_CHUNK,), jnp.int32),
            pltpu.VMEM((_CHUNK,), jnp.int32),
            pltpu.VMEM((_CHUNK, _W), jnp.float32),
            pltpu.VMEM((_CHUNK, d), jnp.float32),
            pltpu.VMEM_SHARED((_NPAD, d), jnp.float32),
            pltpu.SemaphoreType.DMA,
        ],
        compiler_params=pltpu.CompilerParams(use_tc_tiling_on_sc=False),
    )
    zeros = jnp.zeros((_NPAD, d), jnp.float32)
    return kfn(src_p, dst_p, ee, h, zeros)


def _dinv_body(d0_ref, d1_ref, out_ref):
    out_ref[...] = 1.0 / (d0_ref[...] + d1_ref[...] + 1e-16)


def _dinv_kernel(den):
    d0 = den[:_NPAD]
    d1 = den[_NPAD:]
    return pl.pallas_call(
        _dinv_body,
        out_shape=jax.ShapeDtypeStruct((_NPAD, _W), jnp.float32),
    )(d0, d1)


def _block_diag_a(a):
    ch = a.shape[1]
    out = jnp.zeros((HEADS * ch, HEADS), jnp.float32)
    for hh in range(HEADS):
        out = out.at[hh * ch:(hh + 1) * ch, hh].set(a[hh])
    return out


def kernel(x, edge_index, edge_attr, W1, as1, ad1, b1, W2, as2, ad2, b2,
           W3, as3, ad3, b3, W4, as4, ad4, b4, Wf1, bf1, Wf2, bf2, Wf3, bf3):
    n = x.shape[0]
    loop = jnp.arange(n, dtype=edge_index.dtype)
    n_real = edge_index.shape[1] + n
    pad = _E_PAD - n_real
    src_p = jnp.concatenate(
        [edge_index[0], loop, jnp.zeros((pad,), edge_index.dtype)])
    dst_p = jnp.concatenate(
        [edge_index[1], loop, jnp.zeros((pad,), edge_index.dtype)])

    layers = [
        (W1, as1, ad1, b1), (W2, as2, ad2, b2),
        (W3, as3, ad3, b3), (W4, as4, ad4, b4),
    ]
    h_in = x
    for (W, a_s, a_d, b) in layers:
        asrc_bd = _block_diag_a(a_s)
        adst_bd = _block_diag_a(a_d)
        h, es, ed, em = _layer_mm(h_in, W, asrc_bd, adst_bd, n, 2000)
        p = _c_kernel(es, ed, em)
        ee, den = _sc_pass1(src_p, dst_p, p, n_real)
        dinv = _dinv_kernel(den)
        d = h.shape[1]
        ch = d // HEADS
        parts = []
        for lo in range(0, d, 128):
            hi = min(lo + 128, d)
            out2 = _sc_pass2(src_p, dst_p, ee, h[:, lo:hi], ch, lo // ch)
            parts.append(_post_kernel(
                out2[:n], out2[_NPAD:_NPAD + n],
                dinv[:n, lo // ch:hi // ch], b[lo:hi], n, 2000))
        h_in = parts[0] if len(parts) == 1 else jnp.concatenate(parts, axis=1)

    hdim = h_in.shape[1]
    row = _mean_row_kernel(h_in, Wf1[:hdim], bf1)
    z = _mlp_kernel(edge_attr, row, Wf1[hdim:], Wf2, bf2, Wf3, bf3, 3200)
    return z


# speedup vs baseline: 1.2392x; 1.2392x over previous
import functools
import jax
import jax.numpy as jnp
import numpy as np
from jax import lax
from jax.experimental import pallas as pl
from jax.experimental.pallas import tpu as pltpu
from jax.experimental.pallas import tpu_sc as plsc

HEADS = 4

_NC, _NS, _LANES = 2, 16, 16
_NTILES = _NC * _NS
_CHUNK = 256
_CHUNKS_PER_TILE = 21
_EDGES_PER_TILE = _CHUNK * _CHUNKS_PER_TILE
_E_PAD = _EDGES_PER_TILE * _NTILES
_NPAD = 10240
_ROWS_PER_TILE = _NPAD // _NS


def _layer_mm_body(x_ref, w_ref, asrc_ref, adst_ref, h_ref, es_ref, ed_ref,
                   em_ref):
    h = jnp.dot(x_ref[...], w_ref[...], preferred_element_type=jnp.float32)
    h_ref[...] = h
    es = jnp.dot(h, asrc_ref[...], preferred_element_type=jnp.float32)
    ed = jnp.dot(h, adst_ref[...], preferred_element_type=jnp.float32)
    es_ref[...] = es
    ed_ref[...] = ed

    @pl.when(pl.program_id(0) == 0)
    def _():
        em_ref[...] = jnp.full_like(em_ref[...], -jnp.inf)

    em_ref[...] = jnp.maximum(em_ref[...], jnp.max(es, axis=0, keepdims=True))


def _layer_mm(x, W, asrc_bd, adst_bd, n_rows, row_blk):
    din = x.shape[1]
    dout = W.shape[1]
    grid = n_rows // row_blk
    return pl.pallas_call(
        _layer_mm_body,
        grid=(grid,),
        in_specs=[
            pl.BlockSpec((row_blk, din), lambda i: (i, 0)),
            pl.BlockSpec((din, dout), lambda i: (0, 0)),
            pl.BlockSpec((dout, HEADS), lambda i: (0, 0)),
            pl.BlockSpec((dout, HEADS), lambda i: (0, 0)),
        ],
        out_specs=[
            pl.BlockSpec((row_blk, dout), lambda i: (i, 0)),
            pl.BlockSpec((row_blk, HEADS), lambda i: (i, 0)),
            pl.BlockSpec((row_blk, HEADS), lambda i: (i, 0)),
            pl.BlockSpec((1, HEADS), lambda i: (0, 0)),
        ],
        out_shape=[
            jax.ShapeDtypeStruct((n_rows, dout), jnp.float32),
            jax.ShapeDtypeStruct((n_rows, HEADS), jnp.float32),
            jax.ShapeDtypeStruct((n_rows, HEADS), jnp.float32),
            jax.ShapeDtypeStruct((1, HEADS), jnp.float32),
        ],
    )(x, W, asrc_bd, adst_bd)


def _leaky(x):
    return jnp.where(x > 0, x, 0.2 * x)


def _c_body(es_ref, ed_ref, em_ref, p_ref):
    es = es_ref[...]
    ed = ed_ref[...]
    t = em_ref[0, :][None, :] + ed
    u = es + ed
    c = jnp.maximum(_leaky(t), _leaky(u))
    z = jnp.zeros_like(es)
    p_ref[...] = jnp.concatenate([es, ed, c, z], axis=1)


def _c_kernel(es, ed, em):
    n = es.shape[0]
    return pl.pallas_call(
        _c_body,
        out_shape=jax.ShapeDtypeStruct((n, 4 * HEADS), jnp.float32),
    )(es, ed, em)


def _post_body(nheads, p0_ref, p1_ref, dinv_ref, b_ref, out_ref):
    ch = p0_ref.shape[1] // nheads
    scale = jnp.repeat(dinv_ref[...], ch, axis=1)
    v = (p0_ref[...] + p1_ref[...]) * scale + b_ref[...]
    out_ref[...] = jnp.where(v > 0, v, jnp.exp(jnp.minimum(v, 0.0)) - 1.0)


def _post_kernel(p0, p1, dinv, b, n_rows, row_blk):
    d = p0.shape[1]
    nheads = dinv.shape[1]
    grid = n_rows // row_blk
    return pl.pallas_call(
        functools.partial(_post_body, nheads),
        grid=(grid,),
        in_specs=[
            pl.BlockSpec((row_blk, d), lambda i: (i, 0)),
            pl.BlockSpec((row_blk, d), lambda i: (i, 0)),
            pl.BlockSpec((row_blk, nheads), lambda i: (i, 0)),
            pl.BlockSpec((1, d), lambda i: (0, 0)),
        ],
        out_specs=pl.BlockSpec((row_blk, d), lambda i: (i, 0)),
        out_shape=jax.ShapeDtypeStruct((n_rows, d), jnp.float32),
    )(p0, p1, dinv, b.reshape(1, d))


def _mean_row_body(h_ref, wf1a_ref, bf1_ref, row_ref):
    hm = jnp.mean(h_ref[...], axis=0, keepdims=True)
    row_ref[...] = (
        jnp.dot(hm, wf1a_ref[...], preferred_element_type=jnp.float32)
        + bf1_ref[...]
    )


def _mean_row_kernel(h, wf1a, bf1):
    f = wf1a.shape[1]
    return pl.pallas_call(
        _mean_row_body,
        out_shape=jax.ShapeDtypeStruct((1, f), jnp.float32),
    )(h, wf1a, bf1.reshape(1, f))


def _mlp_body(ea_ref, row_ref, wf1b_ref, wf2_ref, bf2_ref, wf3_ref, bf3_ref,
              out_ref):
    z = jnp.dot(ea_ref[...], wf1b_ref[...], preferred_element_type=jnp.float32)
    z = jnp.maximum(z + row_ref[...], 0.0)
    z = jnp.dot(z, wf2_ref[...], preferred_element_type=jnp.float32)
    z = jnp.maximum(z + bf2_ref[...], 0.0)
    z = jnp.dot(z, wf3_ref[...], preferred_element_type=jnp.float32)
    out_ref[...] = z + bf3_ref[...]


def _mlp_kernel(ea, row, wf1b, wf2, bf2, wf3, bf3, row_blk):
    e, de = ea.shape
    f1 = row.shape[1]
    f2 = wf2.shape[1]
    grid = e // row_blk
    return pl.pallas_call(
        _mlp_body,
        grid=(grid,),
        in_specs=[
            pl.BlockSpec((row_blk, de), lambda i: (i, 0)),
            pl.BlockSpec((1, f1), lambda i: (0, 0)),
            pl.BlockSpec((de, f1), lambda i: (0, 0)),
            pl.BlockSpec((f1, f2), lambda i: (0, 0)),
            pl.BlockSpec((1, f2), lambda i: (0, 0)),
            pl.BlockSpec((f2, 1), lambda i: (0, 0)),
            pl.BlockSpec((1, 1), lambda i: (0, 0)),
        ],
        out_specs=pl.BlockSpec((row_blk, 1), lambda i: (i, 0)),
        out_shape=jax.ShapeDtypeStruct((e, 1), jnp.float32),
    )(ea, row, wf1b, wf2, bf2.reshape(1, f2), wf3, bf3.reshape(1, 1))


def _sc_mesh():
    return plsc.VectorSubcoreMesh(core_axis_name="c", subcore_axis_name="s")


_W = 4 * HEADS


def _vbcast(vec, k):
    idx = (lax.iota(jnp.int32, _LANES) >> 4) + k
    return vec.at[idx].get(mode="promise_in_bounds", unique_indices=False)


def _pass1_body(n_real_edges, src_h, dst_h, p_h, zeros_h,
                ee_h, den_h,
                idx_s, idx_d, psr, pdr, eer, den_sh, sem):
    cid = lax.axis_index("c")
    sid = lax.axis_index("s")
    wid = cid * _NS + sid
    tile_base = wid * _EDGES_PER_TILE

    @pl.when(sid == 0)
    def _():
        pltpu.sync_copy(zeros_h, den_sh)

    plsc.subcore_barrier()

    lanes = lax.iota(jnp.int32, _LANES)
    head = lanes & 3
    ed_idx = head + 4
    c_idx = head + 8

    def chunk(j, carry):
        base = tile_base + j * _CHUNK
        pltpu.sync_copy(src_h.at[pl.ds(base, _CHUNK)], idx_s)
        pltpu.sync_copy(dst_h.at[pl.ds(base, _CHUNK)], idx_d)
        pltpu.async_copy(p_h.at[idx_s], psr, sem).wait()
        pltpu.async_copy(p_h.at[idx_d], pdr, sem).wait()

        def edge4(gg, c2):
            for uu in range(4):
                e = gg * 4 + uu
                a = psr[e, :]
                b = pdr[e, :]
                ed_al = b.at[ed_idx].get(mode="promise_in_bounds")
                c_al = b.at[c_idx].get(mode="promise_in_bounds")
                u = a + ed_al
                lr = jnp.where(u > 0, u, 0.2 * u)
                t = jnp.exp(lr - c_al)
                m4 = jnp.where(lanes < 4, 1.0, 0.0)
                g = jnp.where(base + e < n_real_edges, 1.0, 0.0)
                eer[e, :] = t * (m4 * g)
            return c2

        lax.fori_loop(0, _CHUNK // 4, edge4, 0)
        pltpu.sync_copy(eer, ee_h.at[pl.ds(base, _CHUNK)])
        pltpu.sync_copy(eer, den_sh.at[idx_d], add=True)
        return carry

    lax.fori_loop(0, _CHUNKS_PER_TILE, chunk, 0)
    plsc.subcore_barrier()
    row0 = sid * _ROWS_PER_TILE
    pltpu.sync_copy(den_sh.at[pl.ds(row0, _ROWS_PER_TILE)],
                    den_h.at[pl.ds(cid * _NPAD + row0, _ROWS_PER_TILE)])


def _sc_pass1(src_p, dst_p, p, n_real_edges):
    kfn = pl.kernel(
        functools.partial(_pass1_body, n_real_edges),
        mesh=_sc_mesh(),
        out_type=[
            jax.ShapeDtypeStruct((_E_PAD, _W), jnp.float32),
            jax.ShapeDtypeStruct((_NC * _NPAD, _W), jnp.float32),
        ],
        scratch_types=[
            pltpu.VMEM((_CHUNK,), jnp.int32),
            pltpu.VMEM((_CHUNK,), jnp.int32),
            pltpu.VMEM((_CHUNK, _W), jnp.float32),
            pltpu.VMEM((_CHUNK, _W), jnp.float32),
            pltpu.VMEM((_CHUNK, _W), jnp.float32),
            pltpu.VMEM_SHARED((_NPAD, _W), jnp.float32),
            pltpu.SemaphoreType.DMA,
        ],
        compiler_params=pltpu.CompilerParams(use_tc_tiling_on_sc=False),
    )
    zeros = jnp.zeros((_NPAD, _W), jnp.float32)
    return kfn(src_p, dst_p, p, zeros)


_P2CHUNK = 128
_P2CHUNKS = _EDGES_PER_TILE // _P2CHUNK


def _pass2_body(ch, head_base, src_h, dst_h, ee_h, h_h, zeros_h,
                out_h,
                idx_s0, idx_d0, eer0, hr0, idx_s1, idx_d1, eer1, hr1,
                acc_sh, sem0, sem1):
    cid = lax.axis_index("c")
    sid = lax.axis_index("s")
    wid = cid * _NS + sid
    tile_base = wid * _EDGES_PER_TILE
    d = hr0.shape[1]
    vregs_per_row = d // _LANES
    bufs = ((idx_s0, idx_d0, eer0, hr0, sem0),
            (idx_s1, idx_d1, eer1, hr1, sem1))

    @pl.when(sid == 0)
    def _():
        pltpu.sync_copy(zeros_h, acc_sh)

    plsc.subcore_barrier()

    def issue(j, b):
        idx_s, idx_d, eer, hr, sem = bufs[b]
        base = tile_base + j * _P2CHUNK
        pltpu.sync_copy(src_h.at[pl.ds(base, _P2CHUNK)], idx_s)
        pltpu.sync_copy(dst_h.at[pl.ds(base, _P2CHUNK)], idx_d)
        pltpu.sync_copy(ee_h.at[pl.ds(base, _P2CHUNK)], eer)
        return pltpu.async_copy(h_h.at[idx_s], hr, sem)

    def compute_scatter(b):
        idx_s, idx_d, eer, hr, sem = bufs[b]

        def edge(e, c2):
            arow = eer[e, :]
            bcs = {}
            for jv in range(vregs_per_row):
                hd = head_base + (jv * _LANES) // ch
                if hd not in bcs:
                    bcs[hd] = _vbcast(arow, hd)
                sl = pl.ds(jv * _LANES, _LANES)
                hr[e, sl] = hr[e, sl] * bcs[hd]
            return c2

        lax.fori_loop(0, _P2CHUNK, edge, 0)
        pltpu.sync_copy(hr, acc_sh.at[idx_d], add=True)

    pend = issue(0, 0)
    for j in range(_P2CHUNKS):
        b = j % 2
        nxt = issue(j + 1, 1 - b) if j + 1 < _P2CHUNKS else None
        pend.wait()
        compute_scatter(b)
        pend = nxt

    plsc.subcore_barrier()
    row0 = sid * _ROWS_PER_TILE
    pltpu.sync_copy(acc_sh.at[pl.ds(row0, _ROWS_PER_TILE)],
                    out_h.at[pl.ds(cid * _NPAD + row0, _ROWS_PER_TILE)])


def _sc_pass2(src_p, dst_p, ee, h, ch, head_base):
    d = h.shape[1]
    kfn = pl.kernel(
        functools.partial(_pass2_body, ch, head_base),
        mesh=_sc_mesh(),
        out_type=jax.ShapeDtypeStruct((_NC * _NPAD, d), jnp.float32),
        scratch_types=[
            pltpu.VMEM((_P2CHUNK,), jnp.int32),
            pltpu.VMEM((_P2CHUNK,), jnp.int32),
            pltpu.VMEM((_P2CHUNK, _W), jnp.float32),
            pltpu.VMEM((_P2CHUNK, d), jnp.float32),
            pltpu.VMEM((_P2CHUNK,), jnp.int32),
            pltpu.VMEM((_P2CHUNK,), jnp.int32),
            pltpu.VMEM((_P2CHUNK, _W), jnp.float32),
            pltpu.VMEM((_P2CHUNK, d), jnp.float32),
            pltpu.VMEM_SHARED((_NPAD, d), jnp.float32),
            pltpu.SemaphoreType.DMA,
            pltpu.SemaphoreType.DMA,
        ],
        compiler_params=pltpu.CompilerParams(use_tc_tiling_on_sc=False),
    )
    zeros = jnp.zeros((_NPAD, d), jnp.float32)
    return kfn(src_p, dst_p, ee, h, zeros)


def _dinv_body(d0_ref, d1_ref, out_ref):
    out_ref[...] = 1.0 / (d0_ref[...] + d1_ref[...] + 1e-16)


def _dinv_kernel(den):
    d0 = den[:_NPAD]
    d1 = den[_NPAD:]
    return pl.pallas_call(
        _dinv_body,
        out_shape=jax.ShapeDtypeStruct((_NPAD, _W), jnp.float32),
    )(d0, d1)


def _block_diag_a(a):
    ch = a.shape[1]
    out = jnp.zeros((HEADS * ch, HEADS), jnp.float32)
    for hh in range(HEADS):
        out = out.at[hh * ch:(hh + 1) * ch, hh].set(a[hh])
    return out


def kernel(x, edge_index, edge_attr, W1, as1, ad1, b1, W2, as2, ad2, b2,
           W3, as3, ad3, b3, W4, as4, ad4, b4, Wf1, bf1, Wf2, bf2, Wf3, bf3):
    n = x.shape[0]
    loop = jnp.arange(n, dtype=edge_index.dtype)
    n_real = edge_index.shape[1] + n
    pad = _E_PAD - n_real
    src_p = jnp.concatenate(
        [edge_index[0], loop, jnp.zeros((pad,), edge_index.dtype)])
    dst_p = jnp.concatenate(
        [edge_index[1], loop, jnp.zeros((pad,), edge_index.dtype)])

    layers = [
        (W1, as1, ad1, b1), (W2, as2, ad2, b2),
        (W3, as3, ad3, b3), (W4, as4, ad4, b4),
    ]
    h_in = x
    for (W, a_s, a_d, b) in layers:
        asrc_bd = _block_diag_a(a_s)
        adst_bd = _block_diag_a(a_d)
        h, es, ed, em = _layer_mm(h_in, W, asrc_bd, adst_bd, n, 2000)
        p = _c_kernel(es, ed, em)
        ee, den = _sc_pass1(src_p, dst_p, p, n_real)
        dinv = _dinv_kernel(den)
        d = h.shape[1]
        ch = d // HEADS
        parts = []
        for lo in range(0, d, 128):
            hi = min(lo + 128, d)
            out2 = _sc_pass2(src_p, dst_p, ee, h[:, lo:hi], ch, lo // ch)
            parts.append(_post_kernel(
                out2[:n], out2[_NPAD:_NPAD + n],
                dinv[:n, lo // ch:hi // ch], b[lo:hi], n, 2000))
        h_in = parts[0] if len(parts) == 1 else jnp.concatenate(parts, axis=1)

    hdim = h_in.shape[1]
    row = _mean_row_kernel(h_in, Wf1[:hdim], bf1)
    z = _mlp_kernel(edge_attr, row, Wf1[hdim:], Wf2, bf2, Wf3, bf3, 3200)
    return z
